# Initial kernel scaffold; baseline (speedup 1.0000x reference)
import jax
import jax.numpy as jnp
from jax.experimental import pallas as pl


def kernel(spikes, velocities, wp_t, bp, wih_t, whh_t, bg, wo_t, bo):
    raise NotImplementedError("write your pallas kernel here")


# speedup vs baseline: 1.0330x; 1.0330x over previous
import functools

import jax
import jax.numpy as jnp
from jax.experimental import pallas as pl
from jax.experimental.pallas import tpu as pltpu


def _lstm_kernel(x_ref, wx_ref, bx_ref, whh_ref, wo_ref, bo_ref,
                 out_ref, gx_scr, h_scr, *, seq_len, batch_blk):
    T, Bc = seq_len, batch_blk
    H = whh_ref.shape[0]
    D = x_ref.shape[-1]

    x_all = x_ref[...].reshape(T * Bc, D)
    gx_scr[...] = (jnp.dot(x_all, wx_ref[...],
                           preferred_element_type=jnp.float32)
                   + bx_ref[...])

    whh = whh_ref[...]
    h = jnp.zeros((Bc, H), jnp.float32)
    c = jnp.zeros((Bc, H), jnp.float32)
    for t in range(T):
        r0 = t * Bc
        g = gx_scr[r0:r0 + Bc, :] + jnp.dot(
            h, whh, preferred_element_type=jnp.float32)
        i_g = jax.nn.sigmoid(g[:, :H])
        f_g = jax.nn.sigmoid(g[:, H:2 * H])
        g_g = jnp.tanh(g[:, 2 * H:3 * H])
        o_g = jax.nn.sigmoid(g[:, 3 * H:])
        c = f_g * c + i_g * g_g
        h = o_g * jnp.tanh(c)
        h_scr[r0:r0 + Bc, :] = h

    out_ref[...] = (jnp.dot(h_scr[...], wo_ref[...],
                            preferred_element_type=jnp.float32)
                    + bo_ref[...]).reshape(T, Bc, -1)


def kernel(spikes, velocities, wp_t, bp, wih_t, whh_t, bg, wo_t, bo):
    B, T, n_neurons = spikes.shape
    D_in = wp_t.shape[0]
    H = wp_t.shape[1]
    n_out = wo_t.shape[1]
    n_fr_bins = n_out // n_neurons

    wx = wp_t @ wih_t
    bx = bp @ wih_t + bg

    x = jnp.concatenate([spikes, velocities.reshape(B, T, -1)], axis=2)
    x_tm = jnp.transpose(x, (1, 0, 2)).astype(jnp.float32)

    n_cores = 2 if B % 16 == 0 else 1
    Bc = B // n_cores

    kfn = functools.partial(_lstm_kernel, seq_len=T, batch_blk=Bc)

    out = pl.pallas_call(
        kfn,
        out_shape=jax.ShapeDtypeStruct((T, B, n_out), jnp.float32),
        grid=(n_cores,),
        in_specs=[
            pl.BlockSpec((T, Bc, D_in), lambda i: (0, i, 0)),
            pl.BlockSpec((D_in, 4 * H), lambda i: (0, 0)),
            pl.BlockSpec((1, 4 * H), lambda i: (0, 0)),
            pl.BlockSpec((H, 4 * H), lambda i: (0, 0)),
            pl.BlockSpec((H, n_out), lambda i: (0, 0)),
            pl.BlockSpec((1, n_out), lambda i: (0, 0)),
        ],
        out_specs=pl.BlockSpec((T, Bc, n_out), lambda i: (0, i, 0)),
        scratch_shapes=[
            pltpu.VMEM((T * Bc, 4 * H), jnp.float32),
            pltpu.VMEM((T * Bc, H), jnp.float32),
        ],
        compiler_params=pltpu.CompilerParams(
            dimension_semantics=("parallel",)),
    )(x_tm, wx, bx, whh_t, wo_t, bo)

    out = jnp.transpose(out, (1, 0, 2))
    return out.reshape(B, T, n_neurons, n_fr_bins)
